# Initial kernel scaffold; baseline (speedup 1.0000x reference)
import jax
import jax.numpy as jnp
from jax.experimental import pallas as pl


def kernel(input, cache, slot_mapping):
    raise NotImplementedError("write your pallas kernel here")


# speedup vs baseline: 1.6582x; 1.6582x over previous
import functools

import jax
import jax.numpy as jnp
from jax import lax
from jax.experimental import pallas as pl
from jax.experimental.pallas import tpu as pltpu
from jax.experimental.pallas import tpu_sc as plsc

NUM_SLOTS = 1024
NUM_ROWS = 32768

NC = 2
NS = 16
NW = NC * NS
RPW = NUM_SLOTS // NW

COPY_BLOCK = 512


def _tc_copy_body(src_ref, dst_ref):
    dst_ref[...] = src_ref[...]


def _tc_copy(cache):
    return pl.pallas_call(
        _tc_copy_body,
        grid=(NUM_ROWS // COPY_BLOCK,),
        in_specs=[pl.BlockSpec((COPY_BLOCK, 8, 128), lambda i: (i, 0, 0))],
        out_specs=pl.BlockSpec((COPY_BLOCK, 8, 128), lambda i: (i, 0, 0)),
        out_shape=jax.ShapeDtypeStruct((NUM_ROWS, 8, 128), cache.dtype),
    )(cache)


_sc_mesh = plsc.VectorSubcoreMesh(core_axis_name="c", subcore_axis_name="s")


@functools.partial(
    pl.kernel,
    mesh=_sc_mesh,
    scratch_types=[
        pltpu.VMEM((RPW,), jnp.int32),
        pltpu.VMEM((RPW, 8, 128), jnp.float32),
        pltpu.SemaphoreType.DMA,
    ],
)
def _sc_scatter(inp_hbm, sm_hbm, out_ref, idx_v, rows_v, sem):
    wid = lax.axis_index("s") * NC + lax.axis_index("c")
    base = wid * RPW
    pltpu.sync_copy(sm_hbm.at[pl.ds(base, RPW)], idx_v)
    pltpu.sync_copy(inp_hbm.at[pl.ds(base, RPW)], rows_v)
    pltpu.async_copy(rows_v, out_ref.at[idx_v], sem).wait()


def kernel(input, cache, slot_mapping):
    out = _tc_copy(cache)
    ref = jax.new_ref(out)
    _sc_scatter(input, slot_mapping.astype(jnp.int32), ref)
    return ref[...]
